# baseline (speedup 1.0000x reference)
import functools

import jax
import jax.numpy as jnp
from jax import lax
from jax.experimental import pallas as pl
from jax.experimental.pallas import tpu as pltpu
from jax.experimental.pallas import tpu_sc as plsc

NC = 2
NS = 16
LANES = 16
RPC = 2
NBUF = 3
SLAB = 128


def _transpose_slab(tb, ob):
    iot = lax.iota(jnp.int32, LANES)
    rows = [iot + ci * LANES for ci in range(SLAB // LANES)]
    zero = lax.bitwise_and(iot, 0)

    @plsc.parallel_loop(0, 64, unroll=8)
    def _cols(j):
        colv = zero + j
        for ci in range(SLAB // LANES):
            vec = tb[j, pl.ds(ci * LANES, LANES)]
            plsc.store_scatter(ob, [rows[ci], colv], vec)


def _tr_body(v, wt_hbm, wtail_hbm, wl_hbm, tbuf, obuf, sem_l, sem_s):
    c = lax.axis_index("c")
    s = lax.axis_index("s")
    wid = c * NS + s
    nw = NC * NS

    n_full = v // SLAB
    n_main = n_full // nw
    n_extra = n_full - n_main * nw
    tail = v - n_full * SLAB

    _load = lambda g, buf: pltpu.async_copy(
        wt_hbm.at[:, pl.ds(g * SLAB, SLAB)], tbuf.at[buf], sem_l)
    _store = lambda g, buf: pltpu.async_copy(
        obuf.at[buf], wl_hbm.at[pl.ds(g * SLAB, SLAB)], sem_s)

    _load(wid, 0)
    _load(nw + wid, 1)

    def _step(t, carry):
        buf = lax.rem(t, 3)
        ob = lax.rem(t, 2)
        g = t * nw + wid
        pltpu.make_async_copy(wt_hbm.at[:, pl.ds(g * SLAB, SLAB)],
                              tbuf.at[buf], sem_l).wait()

        @pl.when(t + 2 < n_main)
        def _():
            _load((t + 2) * nw + wid, lax.rem(t + 2, 3))

        @pl.when(t >= 2)
        def _():
            pltpu.make_async_copy(obuf.at[ob],
                                  wl_hbm.at[pl.ds(0, SLAB)], sem_s).wait()

        _transpose_slab(tbuf.at[buf], obuf.at[ob])
        _store(g, ob)
        return carry

    lax.fori_loop(0, n_main, _step, 0)
    for t in (n_main - 2, n_main - 1):
        pltpu.make_async_copy(obuf.at[t % 2],
                              wl_hbm.at[pl.ds(0, SLAB)], sem_s).wait()

    @pl.when(wid < n_extra)
    def _():
        g = n_main * nw + wid
        pltpu.sync_copy(wt_hbm.at[:, pl.ds(g * SLAB, SLAB)], tbuf.at[0])
        _transpose_slab(tbuf.at[0], obuf.at[0])
        pltpu.sync_copy(obuf.at[0], wl_hbm.at[pl.ds(g * SLAB, SLAB)])

    if tail:
        @pl.when(wid == n_extra)
        def _():
            pltpu.sync_copy(wtail_hbm, obuf.at[1, pl.ds(0, tail)])
            pltpu.sync_copy(obuf.at[1, pl.ds(0, tail)],
                            wl_hbm.at[pl.ds(n_full * SLAB, tail)])


def _gx_body(n_ch, l, d, b_per_w,
             ids_hbm, wl_hbm, out_hbm, ids_v, rows_v, obuf, sem_g):
    c = lax.axis_index("c")
    s = lax.axis_index("s")
    wid = c * NS + s
    nsub = d // LANES
    orows = obuf.shape[0]
    qn = orows // RPC

    pltpu.sync_copy(ids_hbm.at[wid], ids_v)

    for p in range(NBUF - 1):
        pltpu.async_copy(wl_hbm.at[ids_v.at[p]], rows_v.at[p], sem_g)

    def _step(j, carry):
        buf = lax.rem(j, NBUF)
        pltpu.make_async_copy(wl_hbm.at[ids_v.at[j]], rows_v.at[buf],
                              sem_g).wait()

        @pl.when(j + NBUF - 1 < n_ch)
        def _():
            nxt = lax.rem(j + NBUF - 1, NBUF)
            pltpu.async_copy(wl_hbm.at[ids_v.at[j + NBUF - 1]],
                             rows_v.at[nxt], sem_g)

        for r in range(RPC):
            accs = [rows_v[buf, r * l, pl.ds(ci * LANES, LANES)]
                    for ci in range(nsub)]
            for k in range(1, l):
                for ci in range(nsub):
                    accs[ci] = accs[ci] + rows_v[buf, r * l + k,
                                                 pl.ds(ci * LANES, LANES)]
            row = lax.rem(RPC * j, orows) + r
            for ci in range(nsub):
                obuf[row, pl.ds(ci * LANES, LANES)] = accs[ci]

        @pl.when(lax.rem(j, qn) == qn - 1)
        def _():
            q = lax.div(j, qn)
            pltpu.sync_copy(
                obuf, out_hbm.at[pl.ds(wid * b_per_w + q * orows, orows)])

        return carry

    lax.fori_loop(0, n_ch, _step, 0)


def kernel(ids, W):
    b, l = ids.shape
    v, d = W.shape
    nw = NC * NS
    per_w = (b * l) // nw
    ch = RPC * l
    n_ch = per_w // ch
    b_per_w = b // nw

    ids_r = ids.reshape(nw, n_ch, ch)
    wt = jnp.swapaxes(W, 0, 1)
    wtail = jnp.pad(W[(v // SLAB) * SLAB:], ((0, 0), (0, 128 - d)))

    mesh = plsc.VectorSubcoreMesh(core_axis_name="c", subcore_axis_name="s",
                                  num_cores=NC, num_subcores=NS)
    tr = pl.kernel(
        functools.partial(_tr_body, v),
        out_type=jax.ShapeDtypeStruct((v, 128), jnp.float32),
        mesh=mesh,
        compiler_params=pltpu.CompilerParams(needs_layout_passes=False),
        scratch_types=[
            pltpu.VMEM((3, d, SLAB), jnp.float32),
            pltpu.VMEM((2, SLAB, 128), jnp.float32),
            pltpu.SemaphoreType.DMA,
            pltpu.SemaphoreType.DMA,
        ],
    )
    wl = tr(wt, wtail)

    gx = pl.kernel(
        functools.partial(_gx_body, n_ch, l, d, b_per_w),
        out_type=jax.ShapeDtypeStruct((b, d), jnp.float32),
        mesh=mesh,
        scratch_types=[
            pltpu.VMEM((n_ch, ch), jnp.int32),
            pltpu.VMEM((NBUF, ch, 128), jnp.float32),
            pltpu.VMEM((b_per_w // 4, d), jnp.float32),
            pltpu.SemaphoreType.DMA,
        ],
    )
    return gx(ids_r, wl)


# speedup vs baseline: 1.4414x; 1.4414x over previous
import functools

import jax
import jax.numpy as jnp
from jax import lax
from jax.experimental import pallas as pl
from jax.experimental.pallas import tpu as pltpu
from jax.experimental.pallas import tpu_sc as plsc

NC = 2
NS = 16
LANES = 16
RPC = 2
NBUF = 3
SLAB = 128


def _transpose_slab(tb, ob):
    iot = lax.iota(jnp.int32, LANES)
    rows = [iot + ci * LANES for ci in range(SLAB // LANES)]
    zero = lax.bitwise_and(iot, 0)

    @plsc.parallel_loop(0, 64, unroll=8)
    def _cols(j):
        colv = zero + j
        for ci in range(SLAB // LANES):
            vec = tb[j, pl.ds(ci * LANES, LANES)]
            plsc.store_scatter(ob, [rows[ci], colv], vec)


def _tr_body(v, wt_hbm, wtail_hbm, wl_hbm, tbuf, obuf, sem_l, sem_s):
    c = lax.axis_index("c")
    s = lax.axis_index("s")
    wid = c * NS + s
    nw = NC * NS

    n_full = v // SLAB
    n_main = n_full // nw
    n_extra = n_full - n_main * nw
    tail = v - n_full * SLAB

    _load = lambda g, buf: pltpu.async_copy(
        wt_hbm.at[:, pl.ds(g * SLAB, SLAB)], tbuf.at[buf], sem_l)
    _store = lambda g, buf: pltpu.async_copy(
        obuf.at[buf], wl_hbm.at[pl.ds(g * SLAB, SLAB)], sem_s)

    _load(wid, 0)
    _load(nw + wid, 1)

    def _step(t, carry):
        buf = lax.rem(t, 3)
        ob = lax.rem(t, 2)
        g = t * nw + wid
        pltpu.make_async_copy(wt_hbm.at[:, pl.ds(g * SLAB, SLAB)],
                              tbuf.at[buf], sem_l).wait()

        @pl.when(t + 2 < n_main)
        def _():
            _load((t + 2) * nw + wid, lax.rem(t + 2, 3))

        @pl.when(t >= 2)
        def _():
            pltpu.make_async_copy(obuf.at[ob],
                                  wl_hbm.at[pl.ds(0, SLAB)], sem_s).wait()

        _transpose_slab(tbuf.at[buf], obuf.at[ob])
        _store(g, ob)
        return carry

    lax.fori_loop(0, n_main, _step, 0)
    for t in (n_main - 2, n_main - 1):
        pltpu.make_async_copy(obuf.at[t % 2],
                              wl_hbm.at[pl.ds(0, SLAB)], sem_s).wait()

    @pl.when(wid < n_extra)
    def _():
        g = n_main * nw + wid
        pltpu.sync_copy(wt_hbm.at[:, pl.ds(g * SLAB, SLAB)], tbuf.at[0])
        _transpose_slab(tbuf.at[0], obuf.at[0])
        pltpu.sync_copy(obuf.at[0], wl_hbm.at[pl.ds(g * SLAB, SLAB)])

    if tail:
        @pl.when(wid == n_extra)
        def _():
            pltpu.sync_copy(wtail_hbm, obuf.at[1, pl.ds(0, tail)])
            pltpu.sync_copy(obuf.at[1, pl.ds(0, tail)],
                            wl_hbm.at[pl.ds(n_full * SLAB, tail)])


def _gx_body(n_ch, l, d, b_per_w,
             ids_hbm, wl_hbm, out_hbm, ids_v, rows_v, obuf, sem_g):
    c = lax.axis_index("c")
    s = lax.axis_index("s")
    wid = c * NS + s
    nsub = d // LANES
    orows = obuf.shape[0]
    qn = orows // RPC

    pltpu.sync_copy(ids_hbm.at[wid], ids_v)

    for p in range(NBUF - 1):
        pltpu.async_copy(wl_hbm.at[ids_v.at[p]], rows_v.at[p], sem_g)

    def _step(j, carry):
        buf = lax.rem(j, NBUF)
        pltpu.make_async_copy(wl_hbm.at[ids_v.at[j]], rows_v.at[buf],
                              sem_g).wait()

        @pl.when(j + NBUF - 1 < n_ch)
        def _():
            nxt = lax.rem(j + NBUF - 1, NBUF)
            pltpu.async_copy(wl_hbm.at[ids_v.at[j + NBUF - 1]],
                             rows_v.at[nxt], sem_g)

        for r in range(RPC):
            accs = [rows_v[buf, r * l, pl.ds(ci * LANES, LANES)]
                    for ci in range(nsub)]
            for k in range(1, l):
                for ci in range(nsub):
                    accs[ci] = accs[ci] + rows_v[buf, r * l + k,
                                                 pl.ds(ci * LANES, LANES)]
            row = lax.rem(RPC * j, orows) + r
            for ci in range(nsub):
                obuf[row, pl.ds(ci * LANES, LANES)] = accs[ci]

        @pl.when(lax.rem(j, qn) == qn - 1)
        def _():
            q = lax.div(j, qn)
            pltpu.sync_copy(
                obuf, out_hbm.at[pl.ds(wid * b_per_w + q * orows, orows)])

        return carry

    lax.fori_loop(0, n_ch, _step, 0)


def kernel(ids, W):
    b, l = ids.shape
    v, d = W.shape
    nw = NC * NS
    per_w = (b * l) // nw
    ch = RPC * l
    n_ch = per_w // ch
    b_per_w = b // nw

    ids_r = ids.reshape(nw, n_ch, ch)
    wl_pad = jnp.pad(W, ((0, 0), (0, 128 - d)))

    mesh = plsc.VectorSubcoreMesh(core_axis_name="c", subcore_axis_name="s",
                                  num_cores=NC, num_subcores=NS)
    wl = wl_pad

    gx = pl.kernel(
        functools.partial(_gx_body, n_ch, l, d, b_per_w),
        out_type=jax.ShapeDtypeStruct((b, d), jnp.float32),
        mesh=mesh,
        scratch_types=[
            pltpu.VMEM((n_ch, ch), jnp.int32),
            pltpu.VMEM((NBUF, ch, 128), jnp.float32),
            pltpu.VMEM((b_per_w // 4, d), jnp.float32),
            pltpu.SemaphoreType.DMA,
        ],
    )
    return gx(ids_r, wl)


# speedup vs baseline: 1.4710x; 1.0206x over previous
import functools

import jax
import jax.numpy as jnp
from jax import lax
from jax.experimental import pallas as pl
from jax.experimental.pallas import tpu as pltpu
from jax.experimental.pallas import tpu_sc as plsc

NC = 2
NS = 16
LANES = 16
RPC = 2
NBUF = 4


def _sc_body(n_ch, l, d, b_per_w,
             ids_hbm, w_hbm, out_hbm, ids_v, rows_v, obuf, sem_g, sem_o):
    c = lax.axis_index("c")
    s = lax.axis_index("s")
    wid = c * NS + s
    ch = RPC * l
    nsub = d // LANES

    pltpu.sync_copy(ids_hbm.at[wid], ids_v)

    for p in range(NBUF - 1):
        pltpu.async_copy(w_hbm.at[ids_v.at[p]], rows_v.at[p], sem_g)

    def _step(j, carry):
        buf = lax.rem(j, NBUF)
        pltpu.make_async_copy(w_hbm.at[ids_v.at[j]], rows_v.at[buf],
                              sem_g).wait()

        @pl.when(j + NBUF - 1 < n_ch)
        def _():
            nxt = lax.rem(j + NBUF - 1, NBUF)
            pltpu.async_copy(w_hbm.at[ids_v.at[j + NBUF - 1]], rows_v.at[nxt],
                             sem_g)

        for r in range(RPC):
            accs = [rows_v[buf, r * l, pl.ds(ci * LANES, LANES)]
                    for ci in range(nsub)]
            for k in range(1, l):
                for ci in range(nsub):
                    accs[ci] = accs[ci] + rows_v[buf, r * l + k,
                                                 pl.ds(ci * LANES, LANES)]
            for ci in range(nsub):
                obuf[RPC * j + r, pl.ds(ci * LANES, LANES)] = accs[ci]
        return carry

    lax.fori_loop(0, n_ch, _step, 0)
    pltpu.sync_copy(obuf, out_hbm.at[pl.ds(wid * b_per_w, b_per_w)])


def kernel(ids, W):
    b, l = ids.shape
    v, d = W.shape
    nw = NC * NS
    per_w = (b * l) // nw
    ch = RPC * l
    n_ch = per_w // ch
    b_per_w = b // nw

    ids_r = ids.reshape(nw, n_ch, ch)

    mesh = plsc.VectorSubcoreMesh(core_axis_name="c", subcore_axis_name="s",
                                  num_cores=NC, num_subcores=NS)
    run = pl.kernel(
        functools.partial(_sc_body, n_ch, l, d, b_per_w),
        out_type=jax.ShapeDtypeStruct((b, d), jnp.float32),
        mesh=mesh,
        compiler_params=pltpu.CompilerParams(use_tc_tiling_on_sc=False),
        scratch_types=[
            pltpu.VMEM((n_ch, ch), jnp.int32),
            pltpu.VMEM((NBUF, ch, d), jnp.float32),
            pltpu.VMEM((b_per_w, d), jnp.float32),
            pltpu.SemaphoreType.DMA,
            pltpu.SemaphoreType.DMA,
        ],
    )
    return run(ids_r, W)
